# Initial kernel scaffold; baseline (speedup 1.0000x reference)
import jax
import jax.numpy as jnp
from jax.experimental import pallas as pl


def kernel(x, edge_index, edge_w, W, b):
    raise NotImplementedError("write your pallas kernel here")


# speedup vs baseline: 3.6253x; 3.6253x over previous
import functools

import jax
import jax.numpy as jnp
from jax import lax
from jax.experimental import pallas as pl
from jax.experimental.pallas import tpu as pltpu
from jax.experimental.pallas import tpu_sc as plsc

N = 10000
E = 320000
D = 128
ALPHA = 0.5

NC = 2
NS = 16
LANES = 16

EDGES_PER_WORKER = E // (NC * NS)
BATCH = 80
NBATCH = EDGES_PER_WORKER // BATCH
NPAD = 10240
ROWS_PER_SUB = NPAD // NS
ZROWS = 128


def _sc_aggregate_fn():
  mesh = plsc.VectorSubcoreMesh(core_axis_name="c", subcore_axis_name="s")

  @functools.partial(
      pl.kernel,
      out_type=jax.ShapeDtypeStruct((NC, NPAD, D), jnp.float32),
      mesh=mesh,
      compiler_params=pltpu.CompilerParams(needs_layout_passes=False),
      scratch_types=[
          pltpu.VMEM((BATCH,), jnp.int32),
          pltpu.VMEM((BATCH,), jnp.int32),
          pltpu.VMEM((BATCH,), jnp.float32),
          pltpu.VMEM((BATCH, D), jnp.float32),
          pltpu.VMEM((ZROWS, D), jnp.float32),
          pltpu.VMEM_SHARED((NPAD, D), jnp.float32),
          pltpu.SemaphoreType.DMA,
      ],
  )
  def agg(x_hbm, src_hbm, dst_hbm, w_hbm, out_hbm,
          idxs, idxd, wref, rows, zbuf, hsh, sem):
    cid = lax.axis_index("c")
    sid = lax.axis_index("s")

    zeros = jnp.zeros((LANES,), jnp.float32)

    def zrow(r, carry):
      for j in range(D // LANES):
        zbuf[r, pl.ds(j * LANES, LANES)] = zeros
      return carry

    lax.fori_loop(0, ZROWS, zrow, 0)
    for t in range(ROWS_PER_SUB // ZROWS):
      pltpu.sync_copy(zbuf, hsh.at[pl.ds(sid * ROWS_PER_SUB + t * ZROWS, ZROWS)])
    plsc.subcore_barrier()

    wid = cid * NS + sid
    base0 = wid * EDGES_PER_WORKER

    def batch_body(i, carry):
      base = base0 + i * BATCH
      pltpu.sync_copy(src_hbm.at[pl.ds(base, BATCH)], idxs)
      pltpu.sync_copy(dst_hbm.at[pl.ds(base, BATCH)], idxd)
      pltpu.sync_copy(w_hbm.at[pl.ds(base, BATCH)], wref)
      pltpu.async_copy(x_hbm.at[idxs], rows, sem).wait()

      def row_body(r, c2):
        widx = jnp.full((LANES,), r, dtype=jnp.int32)
        ws = 1.0 - plsc.load_gather(wref, [widx])
        for j in range(D // LANES):
          rows[r, pl.ds(j * LANES, LANES)] = rows[r, pl.ds(j * LANES, LANES)] * ws
        return c2

      lax.fori_loop(0, BATCH, row_body, 0)
      pltpu.sync_copy(rows, hsh.at[idxd], add=True)
      return carry

    lax.fori_loop(0, NBATCH, batch_body, 0)
    plsc.subcore_barrier()

    pltpu.sync_copy(
        hsh.at[pl.ds(sid * ROWS_PER_SUB, ROWS_PER_SUB)],
        out_hbm.at[cid, pl.ds(sid * ROWS_PER_SUB, ROWS_PER_SUB)])

  return agg


_sc_aggregate = _sc_aggregate_fn()

BLK = 400


def _tc_linear_body(p_ref, w_ref, b_ref, o_ref):
  h = p_ref[0] + p_ref[1]
  acc = lax.dot_general(h, w_ref[...], (((1,), (1,)), ((), ())),
                        preferred_element_type=jnp.float32)
  o_ref[...] = ALPHA * (acc + b_ref[...])


def _tc_linear(partials, W, b2d):
  return pl.pallas_call(
      _tc_linear_body,
      grid=(N // BLK,),
      in_specs=[
          pl.BlockSpec((NC, BLK, D), lambda i: (0, i, 0)),
          pl.BlockSpec((D, D), lambda i: (0, 0)),
          pl.BlockSpec((1, D), lambda i: (0, 0)),
      ],
      out_specs=pl.BlockSpec((BLK, D), lambda i: (i, 0)),
      out_shape=jax.ShapeDtypeStruct((N, D), jnp.float32),
  )(partials, W, b2d)


@jax.jit
def kernel(x, edge_index, edge_w, W, b):
  src = edge_index[0]
  dst = edge_index[1]
  partials = _sc_aggregate(x, src, dst, edge_w)
  return _tc_linear(partials, W, b.reshape(1, D))


# speedup vs baseline: 6.5972x; 1.8198x over previous
import functools

import jax
import jax.numpy as jnp
from jax import lax
from jax.experimental import pallas as pl
from jax.experimental.pallas import tpu as pltpu
from jax.experimental.pallas import tpu_sc as plsc

N = 10000
E = 320000
D = 128
ALPHA = 0.5

NC = 2
NS = 16
LANES = 16
NW = NC * NS

EDGES_PER_WORKER = E // NW
BATCH = 80
NBATCH = EDGES_PER_WORKER // BATCH
GROUPS = BATCH // LANES
VPR = D // LANES
NPAD = 10240
ROWS_PER_SUB = NPAD // NS


def _sc_aggregate_fn():
  mesh = plsc.VectorSubcoreMesh(core_axis_name="c", subcore_axis_name="s")

  @functools.partial(
      pl.kernel,
      out_type=jax.ShapeDtypeStruct((NC, NPAD, D), jnp.float32),
      mesh=mesh,
      compiler_params=pltpu.CompilerParams(needs_layout_passes=False),
      scratch_types=[
          pltpu.VMEM((BATCH, D), jnp.float32),
          pltpu.VMEM((BATCH, D), jnp.float32),
          pltpu.VMEM((BATCH,), jnp.int32),
          pltpu.VMEM((BATCH,), jnp.int32),
          pltpu.VMEM((BATCH,), jnp.int32),
          pltpu.VMEM((BATCH,), jnp.int32),
          pltpu.VMEM((BATCH,), jnp.float32),
          pltpu.VMEM((BATCH,), jnp.float32),
          pltpu.VMEM((LANES,), jnp.float32),
          pltpu.VMEM_SHARED((NPAD, D), jnp.float32),
          pltpu.SemaphoreType.DMA,
          pltpu.SemaphoreType.DMA,
          pltpu.SemaphoreType.DMA,
          pltpu.SemaphoreType.DMA,
      ],
  )
  def agg(x_hbm, src_hbm, dst_hbm, w_hbm, out_hbm,
          rows0, rows1, schunk0, schunk1, dchunk0, dchunk1, wchunk0, wchunk1,
          wtmp, hsh, gsem0, gsem1, csem0, csem1):
    cid = lax.axis_index("c")
    sid = lax.axis_index("s")
    wid = cid * NS + sid
    base0 = wid * EDGES_PER_WORKER
    bufs = (rows0, rows1)
    schunk = (schunk0, schunk1)
    dchunk = (dchunk0, dchunk1)
    wchunk = (wchunk0, wchunk1)
    gsems = (gsem0, gsem1)
    csems = (csem0, csem1)

    zeros = jnp.zeros((LANES,), jnp.float32)

    def zrow(r, carry):
      for j in range(VPR):
        rows0[r, pl.ds(j * LANES, LANES)] = zeros
      return carry

    lax.fori_loop(0, BATCH, zrow, 0)
    for t in range(ROWS_PER_SUB // BATCH):
      pltpu.sync_copy(rows0, hsh.at[pl.ds(sid * ROWS_PER_SUB + t * BATCH, BATCH)])
    plsc.subcore_barrier()

    def stage(i, b):
      base = base0 + i * BATCH
      pltpu.async_copy(src_hbm.at[pl.ds(base, BATCH)], schunk[b], csems[b])
      pltpu.async_copy(dst_hbm.at[pl.ds(base, BATCH)], dchunk[b], csems[b])
      pltpu.async_copy(w_hbm.at[pl.ds(base, BATCH)], wchunk[b], csems[b])

    def cwait(b):
      pltpu.make_async_copy(src_hbm.at[pl.ds(0, BATCH)], schunk[b], csems[b]).wait()
      pltpu.make_async_copy(dst_hbm.at[pl.ds(0, BATCH)], dchunk[b], csems[b]).wait()
      pltpu.make_async_copy(w_hbm.at[pl.ds(0, BATCH)], wchunk[b], csems[b]).wait()

    def gather(b):
      pltpu.async_copy(x_hbm.at[schunk[b]], bufs[b], gsems[b])

    def gwait(b):
      pltpu.make_async_copy(x_hbm.at[schunk[b]], bufs[b], gsems[b]).wait()

    cidx = [jnp.full((LANES,), rr, dtype=jnp.int32) for rr in range(LANES)]

    def scale(b):
      rows = bufs[b]

      def grp(g, carry):
        for rr in range(LANES):
          r = g * LANES + rr
          widx = jnp.full((LANES,), r, dtype=jnp.int32)
          ws = 1.0 - plsc.load_gather(wchunk[b], [widx])
          for j in range(VPR):
            rows[r, pl.ds(j * LANES, LANES)] = (
                rows[r, pl.ds(j * LANES, LANES)] * ws)
        return carry

      lax.fori_loop(0, GROUPS, grp, 0)

    def scatter(b):
      pltpu.sync_copy(bufs[b], hsh.at[dchunk[b]], add=True)

    stage(0, 0)
    stage(1, 1)
    cwait(0)
    gather(0)

    def pair(p, carry):
      i0 = 2 * p
      cwait(1)
      gather(1)
      gwait(0)
      scale(0)
      scatter(0)
      stage(i0 + 2, 0)
      gwait(1)
      scale(1)
      scatter(1)
      stage(i0 + 3, 1)
      cwait(0)
      gather(0)
      return carry

    lax.fori_loop(0, (NBATCH - 3) // 2, pair, 0)
    cwait(1)
    gather(1)
    gwait(0)
    scale(0)
    scatter(0)
    stage(NBATCH - 1, 0)
    gwait(1)
    scale(1)
    scatter(1)
    cwait(0)
    gather(0)
    gwait(0)
    scale(0)
    scatter(0)
    plsc.subcore_barrier()

    pltpu.sync_copy(
        hsh.at[pl.ds(sid * ROWS_PER_SUB, ROWS_PER_SUB)],
        out_hbm.at[cid, pl.ds(sid * ROWS_PER_SUB, ROWS_PER_SUB)])

  return agg


_sc_aggregate = _sc_aggregate_fn()

BLK = 400


def _tc_linear_body(p_ref, w_ref, b_ref, o_ref):
  h = p_ref[0] + p_ref[1]
  acc = lax.dot_general(h, w_ref[...], (((1,), (1,)), ((), ())),
                        preferred_element_type=jnp.float32)
  o_ref[...] = ALPHA * (acc + b_ref[...])


def _tc_linear(partials, W, b2d):
  return pl.pallas_call(
      _tc_linear_body,
      grid=(N // BLK,),
      in_specs=[
          pl.BlockSpec((NC, BLK, D), lambda i: (0, i, 0)),
          pl.BlockSpec((D, D), lambda i: (0, 0)),
          pl.BlockSpec((1, D), lambda i: (0, 0)),
      ],
      out_specs=pl.BlockSpec((BLK, D), lambda i: (i, 0)),
      out_shape=jax.ShapeDtypeStruct((N, D), jnp.float32),
  )(partials, W, b2d)


@jax.jit
def kernel(x, edge_index, edge_w, W, b):
  src = edge_index[0]
  dst = edge_index[1]
  partials = _sc_aggregate(x, src, dst, edge_w)
  return _tc_linear(partials, W, b.reshape(1, D))
